# Initial kernel scaffold; baseline (speedup 1.0000x reference)
import jax
import jax.numpy as jnp
from jax.experimental import pallas as pl


def kernel(x, W1, b1, W2, b2, W3, b3, W4, b4, codebooks):
    raise NotImplementedError("write your pallas kernel here")


# speedup vs baseline: 1.1350x; 1.1350x over previous
import jax
import jax.numpy as jnp
from jax.experimental import pallas as pl

_N_TOKENS = 16384
_INPUT_DIM = 256
_HIDDEN = 64
_K = 1024
_S = 3
_TN = 512


def _rvq_body(x_ref, w1_ref, b1_ref, w2_ref, b2_ref, w3_ref, b3_ref,
              w4_ref, b4_ref, cbt_ref, hi_ref, mid_ref, lo_ref,
              q_ref, idx_ref, rec_ref):
    f32 = jnp.float32
    x = x_ref[...]
    h1 = jnp.maximum(jnp.dot(x, w1_ref[...], preferred_element_type=f32)
                     + b1_ref[...], 0.0)
    h = jnp.dot(h1, w2_ref[...], preferred_element_type=f32) + b2_ref[...]

    r = h
    q = jnp.zeros_like(h)
    iota = jax.lax.broadcasted_iota(jnp.int32, (_TN, _K), 1)
    for s in range(_S):
        cbt = cbt_ref[s]
        cn = jnp.sum(cbt * cbt, axis=0, keepdims=True)
        rn = jnp.sum(r * r, axis=1, keepdims=True)
        ab = jnp.dot(r, cbt, preferred_element_type=f32)
        d = jnp.sqrt(jnp.maximum((rn + cn) - 2.0 * ab, 0.0))
        m = jnp.min(d, axis=1, keepdims=True)
        idx = jnp.min(jnp.where(d == m, iota, _K), axis=1, keepdims=True)
        idx_ref[:, s:s + 1] = idx
        oh = (iota == idx).astype(jnp.bfloat16)
        sel = (jnp.dot(oh, hi_ref[s], preferred_element_type=f32)
               + jnp.dot(oh, mid_ref[s], preferred_element_type=f32)
               + jnp.dot(oh, lo_ref[s], preferred_element_type=f32))
        q = q + sel
        r = r - sel

    q_ref[...] = q
    d1 = jnp.maximum(jnp.dot(q, w3_ref[...], preferred_element_type=f32)
                     + b3_ref[...], 0.0)
    rec_ref[...] = jnp.dot(d1, w4_ref[...], preferred_element_type=f32) \
        + b4_ref[...]


def kernel(x, W1, b1, W2, b2, W3, b3, W4, b4, codebooks):
    f32 = jnp.float32
    cbt = jnp.swapaxes(codebooks, 1, 2)
    hi = codebooks.astype(jnp.bfloat16)
    rem1 = codebooks - hi.astype(f32)
    mid = rem1.astype(jnp.bfloat16)
    lo = (rem1 - mid.astype(f32)).astype(jnp.bfloat16)

    grid = (_N_TOKENS // _TN,)
    full = lambda shape: pl.BlockSpec(shape, lambda i: (0,) * len(shape))
    q, idx, rec = pl.pallas_call(
        _rvq_body,
        grid=grid,
        in_specs=[
            pl.BlockSpec((_TN, _INPUT_DIM), lambda i: (i, 0)),
            full((_INPUT_DIM, 2 * _HIDDEN)),
            full((1, 2 * _HIDDEN)),
            full((2 * _HIDDEN, _HIDDEN)),
            full((1, _HIDDEN)),
            full((_HIDDEN, 2 * _HIDDEN)),
            full((1, 2 * _HIDDEN)),
            full((2 * _HIDDEN, _INPUT_DIM)),
            full((1, _INPUT_DIM)),
            full((_S, _HIDDEN, _K)),
            full((_S, _K, _HIDDEN)),
            full((_S, _K, _HIDDEN)),
            full((_S, _K, _HIDDEN)),
        ],
        out_specs=[
            pl.BlockSpec((_TN, _HIDDEN), lambda i: (i, 0)),
            pl.BlockSpec((_TN, _S), lambda i: (i, 0)),
            pl.BlockSpec((_TN, _INPUT_DIM), lambda i: (i, 0)),
        ],
        out_shape=[
            jax.ShapeDtypeStruct((_N_TOKENS, _HIDDEN), f32),
            jax.ShapeDtypeStruct((_N_TOKENS, _S), jnp.int32),
            jax.ShapeDtypeStruct((_N_TOKENS, _INPUT_DIM), f32),
        ],
    )(x, W1, b1.reshape(1, -1), W2, b2.reshape(1, -1),
      W3, b3.reshape(1, -1), W4, b4.reshape(1, -1), cbt, hi, mid, lo)
    return (q, idx.T, rec)


# speedup vs baseline: 1.3220x; 1.1647x over previous
import jax
import jax.numpy as jnp
from jax.experimental import pallas as pl

_N_TOKENS = 16384
_INPUT_DIM = 256
_HIDDEN = 64
_K = 1024
_S = 3
_TN = 512


def _rvq_body(x_ref, w1_ref, b1_ref, w2_ref, b2_ref, w3_ref, b3_ref,
              w4_ref, b4_ref, cbt_ref, hi_ref, mid_ref, lo_ref,
              q_ref, idx_ref, rec_ref):
    f32 = jnp.float32
    x = x_ref[...]
    h1 = jnp.maximum(jnp.dot(x, w1_ref[...], preferred_element_type=f32)
                     + b1_ref[...], 0.0)
    h = jnp.dot(h1, w2_ref[...], preferred_element_type=f32) + b2_ref[...]

    r = h
    q = jnp.zeros_like(h)
    iota = jax.lax.broadcasted_iota(jnp.int32, (_TN, _K), 1)
    for s in range(_S):
        cbt = cbt_ref[s]
        cn = 0.5 * jnp.sum(cbt * cbt, axis=0, keepdims=True)
        ab = jnp.dot(r, cbt, preferred_element_type=f32)
        d = cn - ab
        m = jnp.min(d, axis=1, keepdims=True)
        idx = jnp.min(jnp.where(d == m, iota, _K), axis=1, keepdims=True)
        idx_ref[:, s:s + 1] = idx
        oh = (iota == idx).astype(jnp.bfloat16)
        sel = (jnp.dot(oh, hi_ref[s], preferred_element_type=f32)
               + jnp.dot(oh, mid_ref[s], preferred_element_type=f32)
               + jnp.dot(oh, lo_ref[s], preferred_element_type=f32))
        q = q + sel
        r = r - sel

    q_ref[...] = q
    d1 = jnp.maximum(jnp.dot(q, w3_ref[...], preferred_element_type=f32)
                     + b3_ref[...], 0.0)
    rec_ref[...] = jnp.dot(d1, w4_ref[...], preferred_element_type=f32) \
        + b4_ref[...]


def kernel(x, W1, b1, W2, b2, W3, b3, W4, b4, codebooks):
    f32 = jnp.float32
    cbt = jnp.swapaxes(codebooks, 1, 2)
    hi = codebooks.astype(jnp.bfloat16)
    rem1 = codebooks - hi.astype(f32)
    mid = rem1.astype(jnp.bfloat16)
    lo = (rem1 - mid.astype(f32)).astype(jnp.bfloat16)

    grid = (_N_TOKENS // _TN,)
    full = lambda shape: pl.BlockSpec(shape, lambda i: (0,) * len(shape))
    q, idx, rec = pl.pallas_call(
        _rvq_body,
        grid=grid,
        in_specs=[
            pl.BlockSpec((_TN, _INPUT_DIM), lambda i: (i, 0)),
            full((_INPUT_DIM, 2 * _HIDDEN)),
            full((1, 2 * _HIDDEN)),
            full((2 * _HIDDEN, _HIDDEN)),
            full((1, _HIDDEN)),
            full((_HIDDEN, 2 * _HIDDEN)),
            full((1, 2 * _HIDDEN)),
            full((2 * _HIDDEN, _INPUT_DIM)),
            full((1, _INPUT_DIM)),
            full((_S, _HIDDEN, _K)),
            full((_S, _K, _HIDDEN)),
            full((_S, _K, _HIDDEN)),
            full((_S, _K, _HIDDEN)),
        ],
        out_specs=[
            pl.BlockSpec((_TN, _HIDDEN), lambda i: (i, 0)),
            pl.BlockSpec((_TN, _S), lambda i: (i, 0)),
            pl.BlockSpec((_TN, _INPUT_DIM), lambda i: (i, 0)),
        ],
        out_shape=[
            jax.ShapeDtypeStruct((_N_TOKENS, _HIDDEN), f32),
            jax.ShapeDtypeStruct((_N_TOKENS, _S), jnp.int32),
            jax.ShapeDtypeStruct((_N_TOKENS, _INPUT_DIM), f32),
        ],
    )(x, W1, b1.reshape(1, -1), W2, b2.reshape(1, -1),
      W3, b3.reshape(1, -1), W4, b4.reshape(1, -1), cbt, hi, mid, lo)
    return (q, idx.T, rec)
